# baseline (speedup 1.0000x reference)
import functools
import math

import jax
import jax.numpy as jnp
from jax import lax
from jax.experimental import pallas as pl
from jax.experimental.pallas import tpu as pltpu
from jax.experimental.pallas import tpu_sc as plsc

D_MODEL = 64
SCALE = math.sqrt(D_MODEL)

NUM_CORES = 2
NUM_SUBCORES = 16
NUM_WORKERS = NUM_CORES * NUM_SUBCORES

CHUNK = 128


def _make_sc_lookup(batch: int, vocab: int, d: int):
    assert batch % (NUM_WORKERS * CHUNK) == 0
    per_w = batch // NUM_WORKERS
    n_chunks = per_w // CHUNK

    mesh = plsc.VectorSubcoreMesh(core_axis_name="c", subcore_axis_name="s")

    @functools.partial(
        pl.kernel,
        out_type=jax.ShapeDtypeStruct((NUM_WORKERS, n_chunks, CHUNK, d), jnp.float32),
        mesh=mesh,
        scratch_types=[
            pltpu.VMEM((n_chunks, CHUNK), jnp.int32),
            pltpu.VMEM((CHUNK, d), jnp.float32),
            pltpu.SemaphoreType.DMA,
        ],
        compiler_params=pltpu.CompilerParams(use_tc_tiling_on_sc=False),
    )
    def lookup(x_hbm, lut_hbm, out_hbm, idx_v, rows_v, sem):
        w = lax.axis_index("s") * NUM_CORES + lax.axis_index("c")
        pltpu.sync_copy(x_hbm.at[w], idx_v)

        def chunk_body(g, carry):
            pltpu.async_copy(lut_hbm.at[idx_v.at[g]], rows_v, sem).wait()

            def row_body(i, carry2):
                for j in range(d // 16):
                    sl = rows_v[i, pl.ds(j * 16, 16)]
                    rows_v[i, pl.ds(j * 16, 16)] = sl * SCALE
                return carry2

            lax.fori_loop(0, CHUNK, row_body, 0, unroll=4)
            pltpu.sync_copy(rows_v, out_hbm.at[w, g])
            return carry

        lax.fori_loop(0, n_chunks, chunk_body, 0)

    return lookup


def kernel(x, lut):
    b, t = x.shape
    vocab, d = lut.shape
    batch = b * t
    xr = x.reshape(NUM_WORKERS, batch // (NUM_WORKERS * CHUNK), CHUNK)
    xr = xr.astype(jnp.int32)
    out = _make_sc_lookup(batch, vocab, d)(xr, lut)
    return out.reshape(b, t, d)
